# baseline (speedup 1.0000x reference)
import jax
import jax.numpy as jnp
from jax import lax
from jax.experimental import pallas as pl
from jax.experimental.pallas import tpu as pltpu
from jax.experimental.pallas import tpu_sc as plsc

N = 10000
E = 320000
G = 64
K = 80

_MESH = dict(core_axis_name="c", subcore_axis_name="s")


def _worker(c, s):
    return c * 16 + s


def _deg_body(dst_hbm, w_hbm, out0_hbm, out1_hbm, dstv, wv, zb, acc):
    cid = lax.axis_index("c")
    sid = lax.axis_index("s")
    wid = _worker(cid, sid)
    ch = dst_hbm.shape[1]

    @pl.loop(0, 63)
    def _(j):
        zb[pl.ds(j * 16, 16)] = jnp.zeros((16,), jnp.float32)

    @pl.when(sid < 10)
    def _():
        pltpu.sync_copy(zb.at[pl.ds(0, 1000)], acc.at[pl.ds(sid * 1000, 1000)])

    pltpu.sync_copy(dst_hbm.at[wid], dstv)
    pltpu.sync_copy(w_hbm.at[wid], wv)
    plsc.subcore_barrier()

    @pl.loop(0, ch)
    def _(g):
        pltpu.sync_copy(wv.at[g], acc.at[dstv.at[g]], add=True)

    plsc.subcore_barrier()

    @pl.when((sid == 0) & (cid == 0))
    def _():
        pltpu.sync_copy(acc, out0_hbm)

    @pl.when((sid == 0) & (cid == 1))
    def _():
        pltpu.sync_copy(acc, out1_hbm)


def _deg(dst3, w3):
    ch = dst3.shape[1]
    f = pl.kernel(
        _deg_body,
        out_type=[jax.ShapeDtypeStruct((N,), jnp.float32),
                  jax.ShapeDtypeStruct((N,), jnp.float32)],
        mesh=plsc.VectorSubcoreMesh(**_MESH),
        scratch_types=[
            pltpu.VMEM((ch, K), jnp.int32),
            pltpu.VMEM((ch, K), jnp.float32),
            pltpu.VMEM((1008,), jnp.float32),
            pltpu.VMEM_SHARED((N,), jnp.float32),
        ],
    )
    return f(dst3, w3)


def _prop_body(table_hbm, src_hbm, dst_hbm, w_hbm, out_hbm,
               sbuf, dbuf, wbuf, rows, acc, sg, ss, sst):
    cid = lax.axis_index("c")
    sid = lax.axis_index("s")
    nblk = src_hbm.shape[-3]
    blk = src_hbm.shape[-2]
    ch = nblk * blk
    per_core_src = len(src_hbm.shape) == 5
    if per_core_src:
        srcg = src_hbm.at[cid, sid]
        dstg = dst_hbm.at[sid]
        wg = w_hbm.at[sid]
    else:
        wid = _worker(cid, sid)
        srcg = src_hbm.at[wid]
        dstg = dst_hbm.at[wid]
        wg = w_hbm.at[wid]

    @pl.loop(0, K)
    def _(r):
        for v in range(8):
            rows[0, r, pl.ds(v * 16, 16)] = jnp.zeros((16,), jnp.float32)

    for j in range(7):
        pltpu.sync_copy(rows.at[0], acc.at[pl.ds(sid * 624 + j * 80, 80)])
    pltpu.sync_copy(rows.at[0, pl.ds(0, 64)],
                    acc.at[pl.ds(sid * 624 + 560, 64)])

    @pl.when(sid == 0)
    def _():
        pltpu.sync_copy(rows.at[0, pl.ds(0, 16)], acc.at[pl.ds(9984, 16)])

    plsc.subcore_barrier()

    def issue_stage(b, pb):
        pltpu.async_copy(srcg.at[b], sbuf.at[pb], sst)
        pltpu.async_copy(dstg.at[b], dbuf.at[pb], sst)
        pltpu.async_copy(wg.at[b], wbuf.at[pb], sst)

    def wait_stage(b, pb):
        pltpu.make_async_copy(srcg.at[b], sbuf.at[pb], sst).wait()
        pltpu.make_async_copy(dstg.at[b], dbuf.at[pb], sst).wait()
        pltpu.make_async_copy(wg.at[b], wbuf.at[pb], sst).wait()

    issue_stage(0, 0)
    wait_stage(0, 0)
    pltpu.async_copy(table_hbm.at[sbuf.at[0, 0]], rows.at[0], sg.at[0])

    @pl.loop(0, ch, step=2)
    def _(c0):
        for par in (0, 1):
            c = c0 + par
            b = lax.div(c, blk)
            g = lax.rem(c, blk)
            pb = lax.rem(b, 2)

            @pl.when((g == 2) & (b + 1 < nblk))
            def _():
                issue_stage(b + 1, lax.rem(b + 1, 2))

            pltpu.make_async_copy(table_hbm.at[sbuf.at[pb, g]],
                                  rows.at[par], sg.at[par]).wait()

            for q in range(K // 16):
                wb = wbuf[pb, g, pl.ds(q * 16, 16)]
                for j in range(16):
                    ws = wb[j]
                    e = q * 16 + j
                    for v in range(8):
                        sl = pl.ds(v * 16, 16)
                        rows[par, e, sl] = rows[par, e, sl] * ws

            pltpu.async_copy(rows.at[par], acc.at[dbuf.at[pb, g]],
                             ss.at[par], add=True)

            @pl.when(c + 1 < ch)
            def _():
                opar = 1 - par

                @pl.when(c >= 1)
                def _():
                    pltpu.make_async_copy(rows.at[opar],
                                          acc.at[dbuf.at[pb, g]],
                                          ss.at[opar]).wait()

                b1 = lax.div(c + 1, blk)
                g1 = lax.rem(c + 1, blk)
                pb1 = lax.rem(b1, 2)

                @pl.when(g1 == 0)
                def _():
                    wait_stage(b1, pb1)

                pltpu.async_copy(table_hbm.at[sbuf.at[pb1, g1]],
                                 rows.at[opar], sg.at[opar])

    for par in (0, 1):
        pltpu.make_async_copy(rows.at[par], acc.at[dbuf.at[0, 0]],
                              ss.at[par]).wait()

    plsc.subcore_barrier()
    pltpu.sync_copy(acc.at[pl.ds(sid * 624, 624)],
                    out_hbm.at[cid, pl.ds(sid * 624, 624)])

    @pl.when(sid == 0)
    def _():
        pltpu.sync_copy(acc.at[pl.ds(9984, 16)],
                        out_hbm.at[cid, pl.ds(9984, 16)])


def _prop(table, src4, dst4, w4):
    blk = src4.shape[-2]
    f = pl.kernel(
        _prop_body,
        out_type=jax.ShapeDtypeStruct((2, N, 128), jnp.float32),
        mesh=plsc.VectorSubcoreMesh(**_MESH),
        scratch_types=[
            pltpu.VMEM((2, blk, K), jnp.int32),
            pltpu.VMEM((2, blk, K), jnp.int32),
            pltpu.VMEM((2, blk, K), jnp.float32),
            pltpu.VMEM((2, K, 128), jnp.float32),
            pltpu.VMEM_SHARED((N, 128), jnp.float32),
            pltpu.SemaphoreType.DMA((2,)),
            pltpu.SemaphoreType.DMA((2,)),
            pltpu.SemaphoreType.DMA,
        ],
    )
    return f(table, src4, dst4, w4)


def _prep_body(degp_ref, x_ref, dis_ref, xpp_ref):
    deg = jnp.sum(degp_ref[...], axis=0) + 1.0
    dis = lax.rsqrt(deg)[:, None]
    dis_ref[...] = dis
    xpp_ref[...] = x_ref[...] * dis


def _prep(deg_part, x):
    p = deg_part.shape[0]
    return pl.pallas_call(
        _prep_body,
        in_specs=[
            pl.BlockSpec((p, N), lambda: (0, 0)),
            pl.BlockSpec((N, 128), lambda: (0, 0)),
        ],
        out_specs=[
            pl.BlockSpec((N, 1), lambda: (0, 0)),
            pl.BlockSpec((N, 128), lambda: (0, 0)),
        ],
        out_shape=[
            jax.ShapeDtypeStruct((N, 1), jnp.float32),
            jax.ShapeDtypeStruct((N, 128), jnp.float32),
        ],
    )(deg_part, x)


def _l1_body(acc_ref, xpp_ref, dis_ref, w_ref, b_ref, out_ref):
    dis = dis_ref[...]
    p = jnp.sum(acc_ref[...], axis=0) + xpp_ref[...]
    p = p * dis
    h = jnp.dot(p, w_ref[0], preferred_element_type=jnp.float32) + b_ref[0]
    out_ref[0] = jnp.maximum(h, 0.0) * dis


def _layer1(acc, xpp, dis, W0r, b0r):
    p = acc.shape[0]
    return pl.pallas_call(
        _l1_body,
        grid=(5, 2),
        in_specs=[
            pl.BlockSpec((p, 2000, 128), lambda i, c: (0, i, 0)),
            pl.BlockSpec((2000, 128), lambda i, c: (i, 0)),
            pl.BlockSpec((2000, 1), lambda i, c: (i, 0)),
            pl.BlockSpec((1, 128, 128), lambda i, c: (c, 0, 0)),
            pl.BlockSpec((1, 1, 128), lambda i, c: (c, 0, 0)),
        ],
        out_specs=pl.BlockSpec((1, 2000, 128), lambda i, c: (c, i, 0)),
        out_shape=jax.ShapeDtypeStruct((2, N, 128), jnp.float32),
    )(acc, xpp, dis, W0r, b0r)


def _l2_body(acc_ref, h1_ref, dis_ref, w_ref, b_ref, out_ref):
    dis = dis_ref[...]
    m = b_ref[...]
    for c in range(2):
        p = (acc_ref[c] + h1_ref[c]) * dis
        m = m + jnp.dot(p, w_ref[c], preferred_element_type=jnp.float32)
    out_ref[...] = jnp.maximum(m, 0.0)


def _layer2(acc2, h1pp, dis, W1r, b1r):
    return pl.pallas_call(
        _l2_body,
        grid=(5,),
        in_specs=[
            pl.BlockSpec((2, 2000, 128), lambda i: (0, i, 0)),
            pl.BlockSpec((2, 2000, 128), lambda i: (0, i, 0)),
            pl.BlockSpec((2000, 1), lambda i: (i, 0)),
            pl.BlockSpec((2, 128, 384), lambda i: (0, 0, 0)),
            pl.BlockSpec((1, 384), lambda i: (0, 0)),
        ],
        out_specs=pl.BlockSpec((2000, 384), lambda i: (i, 0)),
        out_shape=jax.ShapeDtypeStruct((N, 384), jnp.float32),
    )(acc2, h1pp, dis, W1r, b1r)


def _head_body(h2_ref, batch_ref, wl_ref, bl_ref, w0_ref, b0_ref, w1_ref,
               b1_ref, sums_ref, cnts_ref):
    i = pl.program_id(0)
    h = jnp.maximum(jnp.dot(h2_ref[...], wl_ref[...],
                            preferred_element_type=jnp.float32) + bl_ref[...], 0.0)
    h = jnp.maximum(jnp.dot(h, w0_ref[...],
                            preferred_element_type=jnp.float32) + b0_ref[...], 0.0)
    h = jnp.maximum(jnp.dot(h, w1_ref[...],
                            preferred_element_type=jnp.float32) + b1_ref[...], 0.0)
    seg = lax.broadcasted_iota(jnp.int32, (1, G), 1)
    P = (batch_ref[...] == seg).astype(jnp.float32)
    ps = lax.dot_general(P, h, (((0,), (0,)), ((), ())),
                         preferred_element_type=jnp.float32)
    ones = jnp.ones((h.shape[0], 1), dtype=jnp.float32)
    pc = lax.dot_general(P, ones, (((0,), (0,)), ((), ())),
                         preferred_element_type=jnp.float32)

    @pl.when(i == 0)
    def _():
        sums_ref[...] = ps
        cnts_ref[...] = pc

    @pl.when(i > 0)
    def _():
        sums_ref[...] += ps
        cnts_ref[...] += pc


def _head(h2, batch2d, W_lin, b_lin2, W_l0, b_l02, W_l1p, b_l1p):
    return pl.pallas_call(
        _head_body,
        grid=(5,),
        in_specs=[
            pl.BlockSpec((2000, 384), lambda i: (i, 0)),
            pl.BlockSpec((2000, 1), lambda i: (i, 0)),
            pl.BlockSpec((384, 512), lambda i: (0, 0)),
            pl.BlockSpec((1, 512), lambda i: (0, 0)),
            pl.BlockSpec((512, 256), lambda i: (0, 0)),
            pl.BlockSpec((1, 256), lambda i: (0, 0)),
            pl.BlockSpec((256, 256), lambda i: (0, 0)),
            pl.BlockSpec((1, 256), lambda i: (0, 0)),
        ],
        out_specs=[
            pl.BlockSpec((G, 256), lambda i: (0, 0)),
            pl.BlockSpec((G, 1), lambda i: (0, 0)),
        ],
        out_shape=[
            jax.ShapeDtypeStruct((G, 256), jnp.float32),
            jax.ShapeDtypeStruct((G, 1), jnp.float32),
        ],
    )(h2, batch2d, W_lin, b_lin2, W_l0, b_l02, W_l1p, b_l1p)


def _final_body(sums_ref, cnts_ref, w_ref, b_ref, out_ref):
    pooled = sums_ref[...] / jnp.maximum(cnts_ref[...], 1.0)
    o = jnp.dot(pooled, w_ref[...], preferred_element_type=jnp.float32) + b_ref[...]
    out_ref[...] = jax.nn.sigmoid(o)


def _final(sums, cnts, W_outp, b_out2):
    return pl.pallas_call(
        _final_body,
        in_specs=[
            pl.BlockSpec((G, 256), lambda: (0, 0)),
            pl.BlockSpec((G, 1), lambda: (0, 0)),
            pl.BlockSpec((256, 1), lambda: (0, 0)),
            pl.BlockSpec((1, 1), lambda: (0, 0)),
        ],
        out_specs=pl.BlockSpec((G, 1), lambda: (0, 0)),
        out_shape=jax.ShapeDtypeStruct((G, 1), jnp.float32),
    )(sums, cnts, W_outp, b_out2)


def kernel(x, edge_index, edge_weight, batch, W_conv0, b_conv0, W_conv1,
           b_conv1, W_lin, b_lin, W_l0, b_l0, W_l1, b_l1, W_out, b_out):
    pad = 2560
    src = jnp.concatenate([edge_index[0].astype(jnp.int32),
                           (jnp.arange(pad, dtype=jnp.int32) * 4) % N])
    dst = jnp.concatenate([edge_index[1].astype(jnp.int32),
                           (jnp.arange(pad, dtype=jnp.int32) * 4) % N])
    w = jnp.concatenate([edge_weight, jnp.zeros((pad,), jnp.float32)])

    deg_part = jnp.stack(_deg(dst.reshape(32, 126, K), w.reshape(32, 126, K)))
    dis, xpp = _prep(deg_part, x)

    acc1 = _prop(xpp, src.reshape(32, 9, 14, K), dst.reshape(32, 9, 14, K),
                 w.reshape(32, 9, 14, K))

    W0r = W_conv0.reshape(128, 2, 128).transpose(1, 0, 2)
    b0r = b_conv0.reshape(2, 1, 128)
    h1pp = _layer1(acc1, xpp, dis, W0r, b0r)

    src_l2 = jnp.stack([src, src + N]).reshape(2, 16, 18, 14, K)
    q2 = _prop(h1pp.reshape(2 * N, 128), src_l2,
               dst.reshape(16, 18, 14, K), w.reshape(16, 18, 14, K))

    W1r = W_conv1.reshape(2, 128, 384)
    h2 = _layer2(q2, h1pp, dis, W1r, b_conv1.reshape(1, 384))

    W_l1p = jnp.pad(W_l1, ((0, 0), (0, 86)))
    b_l1p = jnp.pad(b_l1, (0, 86)).reshape(1, 256)
    sums, cnts = _head(h2, batch.astype(jnp.int32).reshape(N, 1), W_lin,
                       b_lin.reshape(1, 512), W_l0, b_l0.reshape(1, 256),
                       W_l1p, b_l1p)

    W_outp = jnp.pad(W_out, ((0, 86), (0, 0)))
    out = _final(sums, cnts, W_outp, b_out.reshape(1, 1))
    return out


# speedup vs baseline: 1.5770x; 1.5770x over previous
import jax
import jax.numpy as jnp
from jax import lax
from jax.experimental import pallas as pl
from jax.experimental.pallas import tpu as pltpu
from jax.experimental.pallas import tpu_sc as plsc

N = 10000
E = 320000
G = 64
K = 80

_MESH = dict(core_axis_name="c", subcore_axis_name="s")


def _worker(c, s):
    return c * 16 + s


def _deg_body(dst_hbm, w_hbm, out0_hbm, out1_hbm, dstv, wv, zb, acc):
    cid = lax.axis_index("c")
    sid = lax.axis_index("s")
    wid = _worker(cid, sid)
    ch = dst_hbm.shape[1]

    @pl.loop(0, 63)
    def _(j):
        zb[pl.ds(j * 16, 16)] = jnp.zeros((16,), jnp.float32)

    @pl.when(sid < 10)
    def _():
        pltpu.sync_copy(zb.at[pl.ds(0, 1000)], acc.at[pl.ds(sid * 1000, 1000)])

    pltpu.sync_copy(dst_hbm.at[wid], dstv)
    pltpu.sync_copy(w_hbm.at[wid], wv)
    plsc.subcore_barrier()

    @pl.loop(0, ch)
    def _(g):
        pltpu.sync_copy(wv.at[g], acc.at[dstv.at[g]], add=True)

    plsc.subcore_barrier()

    @pl.when((sid == 0) & (cid == 0))
    def _():
        pltpu.sync_copy(acc, out0_hbm)

    @pl.when((sid == 0) & (cid == 1))
    def _():
        pltpu.sync_copy(acc, out1_hbm)


def _deg(dst3, w3):
    ch = dst3.shape[1]
    f = pl.kernel(
        _deg_body,
        out_type=[jax.ShapeDtypeStruct((N,), jnp.float32),
                  jax.ShapeDtypeStruct((N,), jnp.float32)],
        mesh=plsc.VectorSubcoreMesh(**_MESH),
        scratch_types=[
            pltpu.VMEM((ch, K), jnp.int32),
            pltpu.VMEM((ch, K), jnp.float32),
            pltpu.VMEM((1008,), jnp.float32),
            pltpu.VMEM_SHARED((N,), jnp.float32),
        ],
    )
    return f(dst3, w3)


def _prop_body(table_hbm, src_hbm, dst_hbm, w_hbm, out_hbm,
               sbuf, dbuf, wbuf, rows, acc, sg, ss, sst):
    cid = lax.axis_index("c")
    sid = lax.axis_index("s")
    nblk = src_hbm.shape[-3]
    blk = src_hbm.shape[-2]
    ch = nblk * blk
    per_core_src = len(src_hbm.shape) == 5
    if per_core_src:
        srcg = src_hbm.at[cid, sid]
        dstg = dst_hbm.at[sid]
        wg = w_hbm.at[sid]
    else:
        wid = _worker(cid, sid)
        srcg = src_hbm.at[wid]
        dstg = dst_hbm.at[wid]
        wg = w_hbm.at[wid]

    @pl.loop(0, K)
    def _(r):
        for v in range(8):
            rows[0, r, pl.ds(v * 16, 16)] = jnp.zeros((16,), jnp.float32)

    for j in range(7):
        pltpu.sync_copy(rows.at[0], acc.at[pl.ds(sid * 624 + j * 80, 80)])
    pltpu.sync_copy(rows.at[0, pl.ds(0, 64)],
                    acc.at[pl.ds(sid * 624 + 560, 64)])

    @pl.when(sid == 0)
    def _():
        pltpu.sync_copy(rows.at[0, pl.ds(0, 16)], acc.at[pl.ds(9984, 16)])

    plsc.subcore_barrier()

    def issue_stage(b, pb):
        pltpu.async_copy(srcg.at[b], sbuf.at[pb], sst)
        pltpu.async_copy(dstg.at[b], dbuf.at[pb], sst)
        pltpu.async_copy(wg.at[b], wbuf.at[pb], sst)

    def wait_stage(b, pb):
        pltpu.make_async_copy(srcg.at[b], sbuf.at[pb], sst).wait()
        pltpu.make_async_copy(dstg.at[b], dbuf.at[pb], sst).wait()
        pltpu.make_async_copy(wg.at[b], wbuf.at[pb], sst).wait()

    issue_stage(0, 0)
    wait_stage(0, 0)
    for c in range(3):
        pltpu.async_copy(table_hbm.at[sbuf.at[0, c]], rows.at[c], sg.at[c])

    @pl.loop(0, ch, step=4)
    def _(c0):
        for par in range(4):
            c = c0 + par
            b = lax.div(c, blk)
            g = lax.rem(c, blk)
            pb = lax.rem(b, 2)

            pltpu.make_async_copy(table_hbm.at[sbuf.at[pb, g]],
                                  rows.at[par], sg.at[par]).wait()

            @pl.loop(0, K // 16)
            def _(q):
                wb = wbuf[pb, g, pl.ds(q * 16, 16)]
                for j in range(16):
                    ws = wb[j]
                    e = q * 16 + j
                    for v in range(8):
                        sl = pl.ds(v * 16, 16)
                        rows[par, e, sl] = rows[par, e, sl] * ws

            pltpu.async_copy(rows.at[par], acc.at[dbuf.at[pb, g]],
                             ss.at[par], add=True)

            cn = c + 3

            @pl.when(cn < ch)
            def _():
                npar = (par + 3) % 4

                @pl.when(c >= 1)
                def _():
                    pltpu.make_async_copy(rows.at[npar],
                                          acc.at[dbuf.at[pb, g]],
                                          ss.at[npar]).wait()

                @pl.when((g == 1) & (b + 1 < nblk))
                def _():
                    issue_stage(b + 1, lax.rem(b + 1, 2))

                b1 = lax.div(cn, blk)
                g1 = lax.rem(cn, blk)
                pb1 = lax.rem(b1, 2)

                @pl.when(g1 == 0)
                def _():
                    wait_stage(b1, pb1)

                pltpu.async_copy(table_hbm.at[sbuf.at[pb1, g1]],
                                 rows.at[npar], sg.at[npar])

    for par in range(4):
        pltpu.make_async_copy(rows.at[par], acc.at[dbuf.at[0, 0]],
                              ss.at[par]).wait()

    plsc.subcore_barrier()
    pltpu.sync_copy(acc.at[pl.ds(sid * 624, 624)],
                    out_hbm.at[cid, pl.ds(sid * 624, 624)])

    @pl.when(sid == 0)
    def _():
        pltpu.sync_copy(acc.at[pl.ds(9984, 16)],
                        out_hbm.at[cid, pl.ds(9984, 16)])


def _prop(table, src4, dst4, w4):
    blk = src4.shape[-2]
    f = pl.kernel(
        _prop_body,
        out_type=jax.ShapeDtypeStruct((2, N, 128), jnp.float32),
        mesh=plsc.VectorSubcoreMesh(**_MESH),
        scratch_types=[
            pltpu.VMEM((2, blk, K), jnp.int32),
            pltpu.VMEM((2, blk, K), jnp.int32),
            pltpu.VMEM((2, blk, K), jnp.float32),
            pltpu.VMEM((4, K, 128), jnp.float32),
            pltpu.VMEM_SHARED((N, 128), jnp.float32),
            pltpu.SemaphoreType.DMA((4,)),
            pltpu.SemaphoreType.DMA((4,)),
            pltpu.SemaphoreType.DMA,
        ],
    )
    return f(table, src4, dst4, w4)


def _prep_body(degp_ref, x_ref, dis_ref, xpp_ref):
    deg = jnp.sum(degp_ref[...], axis=0) + 1.0
    dis = lax.rsqrt(deg)[:, None]
    dis_ref[...] = dis
    xpp_ref[...] = x_ref[...] * dis


def _prep(deg_part, x):
    p = deg_part.shape[0]
    return pl.pallas_call(
        _prep_body,
        in_specs=[
            pl.BlockSpec((p, N), lambda: (0, 0)),
            pl.BlockSpec((N, 128), lambda: (0, 0)),
        ],
        out_specs=[
            pl.BlockSpec((N, 1), lambda: (0, 0)),
            pl.BlockSpec((N, 128), lambda: (0, 0)),
        ],
        out_shape=[
            jax.ShapeDtypeStruct((N, 1), jnp.float32),
            jax.ShapeDtypeStruct((N, 128), jnp.float32),
        ],
    )(deg_part, x)


def _l1_body(acc_ref, xpp_ref, dis_ref, w_ref, b_ref, out_ref):
    dis = dis_ref[...]
    p = jnp.sum(acc_ref[...], axis=0) + xpp_ref[...]
    p = p * dis
    h = jnp.dot(p, w_ref[0], preferred_element_type=jnp.float32) + b_ref[0]
    out_ref[0] = jnp.maximum(h, 0.0) * dis


def _layer1(acc, xpp, dis, W0r, b0r):
    p = acc.shape[0]
    return pl.pallas_call(
        _l1_body,
        grid=(5, 2),
        in_specs=[
            pl.BlockSpec((p, 2000, 128), lambda i, c: (0, i, 0)),
            pl.BlockSpec((2000, 128), lambda i, c: (i, 0)),
            pl.BlockSpec((2000, 1), lambda i, c: (i, 0)),
            pl.BlockSpec((1, 128, 128), lambda i, c: (c, 0, 0)),
            pl.BlockSpec((1, 1, 128), lambda i, c: (c, 0, 0)),
        ],
        out_specs=pl.BlockSpec((1, 2000, 128), lambda i, c: (c, i, 0)),
        out_shape=jax.ShapeDtypeStruct((2, N, 128), jnp.float32),
    )(acc, xpp, dis, W0r, b0r)


def _l2_body(acc_ref, h1_ref, dis_ref, w_ref, b_ref, out_ref):
    dis = dis_ref[...]
    m = b_ref[...]
    for c in range(2):
        p = (acc_ref[c] + h1_ref[c]) * dis
        m = m + jnp.dot(p, w_ref[c], preferred_element_type=jnp.float32)
    out_ref[...] = jnp.maximum(m, 0.0)


def _layer2(acc2, h1pp, dis, W1r, b1r):
    return pl.pallas_call(
        _l2_body,
        grid=(5,),
        in_specs=[
            pl.BlockSpec((2, 2000, 128), lambda i: (0, i, 0)),
            pl.BlockSpec((2, 2000, 128), lambda i: (0, i, 0)),
            pl.BlockSpec((2000, 1), lambda i: (i, 0)),
            pl.BlockSpec((2, 128, 384), lambda i: (0, 0, 0)),
            pl.BlockSpec((1, 384), lambda i: (0, 0)),
        ],
        out_specs=pl.BlockSpec((2000, 384), lambda i: (i, 0)),
        out_shape=jax.ShapeDtypeStruct((N, 384), jnp.float32),
    )(acc2, h1pp, dis, W1r, b1r)


def _head_body(h2_ref, batch_ref, wl_ref, bl_ref, w0_ref, b0_ref, w1_ref,
               b1_ref, sums_ref, cnts_ref):
    i = pl.program_id(0)
    h = jnp.maximum(jnp.dot(h2_ref[...], wl_ref[...],
                            preferred_element_type=jnp.float32) + bl_ref[...], 0.0)
    h = jnp.maximum(jnp.dot(h, w0_ref[...],
                            preferred_element_type=jnp.float32) + b0_ref[...], 0.0)
    h = jnp.maximum(jnp.dot(h, w1_ref[...],
                            preferred_element_type=jnp.float32) + b1_ref[...], 0.0)
    seg = lax.broadcasted_iota(jnp.int32, (1, G), 1)
    P = (batch_ref[...] == seg).astype(jnp.float32)
    ps = lax.dot_general(P, h, (((0,), (0,)), ((), ())),
                         preferred_element_type=jnp.float32)
    ones = jnp.ones((h.shape[0], 1), dtype=jnp.float32)
    pc = lax.dot_general(P, ones, (((0,), (0,)), ((), ())),
                         preferred_element_type=jnp.float32)

    @pl.when(i == 0)
    def _():
        sums_ref[...] = ps
        cnts_ref[...] = pc

    @pl.when(i > 0)
    def _():
        sums_ref[...] += ps
        cnts_ref[...] += pc


def _head(h2, batch2d, W_lin, b_lin2, W_l0, b_l02, W_l1p, b_l1p):
    return pl.pallas_call(
        _head_body,
        grid=(5,),
        in_specs=[
            pl.BlockSpec((2000, 384), lambda i: (i, 0)),
            pl.BlockSpec((2000, 1), lambda i: (i, 0)),
            pl.BlockSpec((384, 512), lambda i: (0, 0)),
            pl.BlockSpec((1, 512), lambda i: (0, 0)),
            pl.BlockSpec((512, 256), lambda i: (0, 0)),
            pl.BlockSpec((1, 256), lambda i: (0, 0)),
            pl.BlockSpec((256, 256), lambda i: (0, 0)),
            pl.BlockSpec((1, 256), lambda i: (0, 0)),
        ],
        out_specs=[
            pl.BlockSpec((G, 256), lambda i: (0, 0)),
            pl.BlockSpec((G, 1), lambda i: (0, 0)),
        ],
        out_shape=[
            jax.ShapeDtypeStruct((G, 256), jnp.float32),
            jax.ShapeDtypeStruct((G, 1), jnp.float32),
        ],
    )(h2, batch2d, W_lin, b_lin2, W_l0, b_l02, W_l1p, b_l1p)


def _final_body(sums_ref, cnts_ref, w_ref, b_ref, out_ref):
    pooled = sums_ref[...] / jnp.maximum(cnts_ref[...], 1.0)
    o = jnp.dot(pooled, w_ref[...], preferred_element_type=jnp.float32) + b_ref[...]
    out_ref[...] = jax.nn.sigmoid(o)


def _final(sums, cnts, W_outp, b_out2):
    return pl.pallas_call(
        _final_body,
        in_specs=[
            pl.BlockSpec((G, 256), lambda: (0, 0)),
            pl.BlockSpec((G, 1), lambda: (0, 0)),
            pl.BlockSpec((256, 1), lambda: (0, 0)),
            pl.BlockSpec((1, 1), lambda: (0, 0)),
        ],
        out_specs=pl.BlockSpec((G, 1), lambda: (0, 0)),
        out_shape=jax.ShapeDtypeStruct((G, 1), jnp.float32),
    )(sums, cnts, W_outp, b_out2)


def kernel(x, edge_index, edge_weight, batch, W_conv0, b_conv0, W_conv1,
           b_conv1, W_lin, b_lin, W_l0, b_l0, W_l1, b_l1, W_out, b_out):
    pad = 7680
    src = jnp.concatenate([edge_index[0].astype(jnp.int32),
                           (jnp.arange(pad, dtype=jnp.int32) * 4) % N])
    dst = jnp.concatenate([edge_index[1].astype(jnp.int32),
                           (jnp.arange(pad, dtype=jnp.int32) * 4) % N])
    w = jnp.concatenate([edge_weight, jnp.zeros((pad,), jnp.float32)])

    deg_part = jnp.stack(_deg(dst.reshape(32, 128, K), w.reshape(32, 128, K)))
    dis, xpp = _prep(deg_part, x)

    acc1 = _prop(xpp, src.reshape(32, 16, 8, K), dst.reshape(32, 16, 8, K),
                 w.reshape(32, 16, 8, K))

    W0r = W_conv0.reshape(128, 2, 128).transpose(1, 0, 2)
    b0r = b_conv0.reshape(2, 1, 128)
    h1pp = _layer1(acc1, xpp, dis, W0r, b0r)

    src_l2 = jnp.stack([src, src + N]).reshape(2, 16, 32, 8, K)
    q2 = _prop(h1pp.reshape(2 * N, 128), src_l2,
               dst.reshape(16, 32, 8, K), w.reshape(16, 32, 8, K))

    W1r = W_conv1.reshape(2, 128, 384)
    h2 = _layer2(q2, h1pp, dis, W1r, b_conv1.reshape(1, 384))

    W_l1p = jnp.pad(W_l1, ((0, 0), (0, 86)))
    b_l1p = jnp.pad(b_l1, (0, 86)).reshape(1, 256)
    sums, cnts = _head(h2, batch.astype(jnp.int32).reshape(N, 1), W_lin,
                       b_lin.reshape(1, 512), W_l0, b_l0.reshape(1, 256),
                       W_l1p, b_l1p)

    W_outp = jnp.pad(W_out, ((0, 86), (0, 0)))
    out = _final(sums, cnts, W_outp, b_out.reshape(1, 1))
    return out


# speedup vs baseline: 1.6286x; 1.0327x over previous
import jax
import jax.numpy as jnp
from jax import lax
from jax.experimental import pallas as pl
from jax.experimental.pallas import tpu as pltpu
from jax.experimental.pallas import tpu_sc as plsc

N = 10000
E = 320000
G = 64
K = 80

_MESH = dict(core_axis_name="c", subcore_axis_name="s")


def _worker(c, s):
    return c * 16 + s


def _deg_body(dst_hbm, w_hbm, out0_hbm, out1_hbm, dstv, wv, zb, acc):
    cid = lax.axis_index("c")
    sid = lax.axis_index("s")
    wid = _worker(cid, sid)
    ch = dst_hbm.shape[1]

    @pl.loop(0, 63)
    def _(j):
        zb[pl.ds(j * 16, 16)] = jnp.zeros((16,), jnp.float32)

    @pl.when(sid < 10)
    def _():
        pltpu.sync_copy(zb.at[pl.ds(0, 1000)], acc.at[pl.ds(sid * 1000, 1000)])

    pltpu.sync_copy(dst_hbm.at[wid], dstv)
    pltpu.sync_copy(w_hbm.at[wid], wv)
    plsc.subcore_barrier()

    @pl.loop(0, ch)
    def _(g):
        pltpu.sync_copy(wv.at[g], acc.at[dstv.at[g]], add=True)

    plsc.subcore_barrier()

    @pl.when((sid == 0) & (cid == 0))
    def _():
        pltpu.sync_copy(acc, out0_hbm)

    @pl.when((sid == 0) & (cid == 1))
    def _():
        pltpu.sync_copy(acc, out1_hbm)


def _deg(dst3, w3):
    ch = dst3.shape[1]
    f = pl.kernel(
        _deg_body,
        out_type=[jax.ShapeDtypeStruct((N,), jnp.float32),
                  jax.ShapeDtypeStruct((N,), jnp.float32)],
        mesh=plsc.VectorSubcoreMesh(**_MESH),
        scratch_types=[
            pltpu.VMEM((ch, K), jnp.int32),
            pltpu.VMEM((ch, K), jnp.float32),
            pltpu.VMEM((1008,), jnp.float32),
            pltpu.VMEM_SHARED((N,), jnp.float32),
        ],
    )
    return f(dst3, w3)


def _prop_body(table_hbm, src_hbm, dst_hbm, w_hbm, out_hbm,
               sbuf, dbuf, wbuf, rows, acc, sg, ss, sst):
    cid = lax.axis_index("c")
    sid = lax.axis_index("s")
    nblk = src_hbm.shape[-3]
    blk = src_hbm.shape[-2]
    ch = nblk * blk
    per_core_src = len(src_hbm.shape) == 5
    if per_core_src:
        srcg = src_hbm.at[cid, sid]
        dstg = dst_hbm.at[sid]
        wg = w_hbm.at[sid]
    else:
        wid = _worker(cid, sid)
        srcg = src_hbm.at[wid]
        dstg = dst_hbm.at[wid]
        wg = w_hbm.at[wid]

    @pl.loop(0, K)
    def _(r):
        for v in range(8):
            rows[0, r, pl.ds(v * 16, 16)] = jnp.zeros((16,), jnp.float32)

    for j in range(7):
        pltpu.sync_copy(rows.at[0], acc.at[pl.ds(sid * 624 + j * 80, 80)])
    pltpu.sync_copy(rows.at[0, pl.ds(0, 64)],
                    acc.at[pl.ds(sid * 624 + 560, 64)])

    @pl.when(sid == 0)
    def _():
        pltpu.sync_copy(rows.at[0, pl.ds(0, 16)], acc.at[pl.ds(9984, 16)])

    plsc.subcore_barrier()

    def issue_stage(b, pb):
        pltpu.async_copy(srcg.at[b], sbuf.at[pb], sst)
        pltpu.async_copy(dstg.at[b], dbuf.at[pb], sst)
        pltpu.async_copy(wg.at[b], wbuf.at[pb], sst)

    def wait_stage(b, pb):
        pltpu.make_async_copy(srcg.at[b], sbuf.at[pb], sst).wait()
        pltpu.make_async_copy(dstg.at[b], dbuf.at[pb], sst).wait()
        pltpu.make_async_copy(wg.at[b], wbuf.at[pb], sst).wait()

    issue_stage(0, 0)
    wait_stage(0, 0)
    for c in range(3):
        pltpu.async_copy(table_hbm.at[sbuf.at[0, c]], rows.at[c], sg.at[c])

    @pl.loop(0, ch, step=4)
    def _(c0):
        for par in range(4):
            c = c0 + par
            b = lax.div(c, blk)
            g = lax.rem(c, blk)
            pb = lax.rem(b, 2)

            pltpu.make_async_copy(table_hbm.at[sbuf.at[pb, g]],
                                  rows.at[par], sg.at[par]).wait()

            @pl.loop(0, K // 16)
            def _(q):
                wb = wbuf[pb, g, pl.ds(q * 16, 16)]
                for j in range(16):
                    ws = wb[j]
                    e = q * 16 + j
                    for v in range(8):
                        sl = pl.ds(v * 16, 16)
                        rows[par, e, sl] = rows[par, e, sl] * ws

            pltpu.async_copy(rows.at[par], acc.at[dbuf.at[pb, g]],
                             ss.at[par], add=True)

            cn = c + 3

            @pl.when(cn < ch)
            def _():
                npar = (par + 3) % 4

                @pl.when(c >= 1)
                def _():
                    pltpu.make_async_copy(rows.at[npar],
                                          acc.at[dbuf.at[pb, g]],
                                          ss.at[npar]).wait()

                @pl.when((g == 1) & (b + 1 < nblk))
                def _():
                    issue_stage(b + 1, lax.rem(b + 1, 2))

                b1 = lax.div(cn, blk)
                g1 = lax.rem(cn, blk)
                pb1 = lax.rem(b1, 2)

                @pl.when(g1 == 0)
                def _():
                    wait_stage(b1, pb1)

                pltpu.async_copy(table_hbm.at[sbuf.at[pb1, g1]],
                                 rows.at[npar], sg.at[npar])

    for par in range(4):
        pltpu.make_async_copy(rows.at[par], acc.at[dbuf.at[0, 0]],
                              ss.at[par]).wait()

    plsc.subcore_barrier()
    pltpu.sync_copy(acc.at[pl.ds(sid * 624, 624)],
                    out_hbm.at[cid, pl.ds(sid * 624, 624)])

    @pl.when(sid == 0)
    def _():
        pltpu.sync_copy(acc.at[pl.ds(9984, 16)],
                        out_hbm.at[cid, pl.ds(9984, 16)])


def _prop(table, src4, dst4, w4):
    blk = src4.shape[-2]
    f = pl.kernel(
        _prop_body,
        out_type=jax.ShapeDtypeStruct((2, N, 128), jnp.float32),
        mesh=plsc.VectorSubcoreMesh(**_MESH),
        scratch_types=[
            pltpu.VMEM((2, blk, K), jnp.int32),
            pltpu.VMEM((2, blk, K), jnp.int32),
            pltpu.VMEM((2, blk, K), jnp.float32),
            pltpu.VMEM((4, K, 128), jnp.float32),
            pltpu.VMEM_SHARED((N, 128), jnp.float32),
            pltpu.SemaphoreType.DMA((4,)),
            pltpu.SemaphoreType.DMA((4,)),
            pltpu.SemaphoreType.DMA,
        ],
    )
    return f(table, src4, dst4, w4)


def _prep_body(degp_ref, x_ref, dis_ref, xpp_ref):
    deg = jnp.sum(degp_ref[...], axis=0) + 1.0
    dis = lax.rsqrt(deg)[:, None]
    dis_ref[...] = dis
    xpp_ref[...] = x_ref[...] * dis


def _prep(deg_part, x):
    p = deg_part.shape[0]
    return pl.pallas_call(
        _prep_body,
        in_specs=[
            pl.BlockSpec((p, N), lambda: (0, 0)),
            pl.BlockSpec((N, 128), lambda: (0, 0)),
        ],
        out_specs=[
            pl.BlockSpec((N, 1), lambda: (0, 0)),
            pl.BlockSpec((N, 128), lambda: (0, 0)),
        ],
        out_shape=[
            jax.ShapeDtypeStruct((N, 1), jnp.float32),
            jax.ShapeDtypeStruct((N, 128), jnp.float32),
        ],
    )(deg_part, x)


def _l1_body(acc_ref, xpp_ref, dis_ref, w_ref, b_ref, out_ref):
    dis = dis_ref[...]
    p = jnp.sum(acc_ref[...], axis=0) + xpp_ref[...]
    p = p * dis
    h = jnp.dot(p, w_ref[0], preferred_element_type=jnp.float32) + b_ref[0]
    out_ref[0] = jnp.maximum(h, 0.0) * dis


def _layer1(acc, xpp, dis, W0r, b0r):
    p = acc.shape[0]
    return pl.pallas_call(
        _l1_body,
        grid=(5, 2),
        in_specs=[
            pl.BlockSpec((p, 2000, 128), lambda i, c: (0, i, 0)),
            pl.BlockSpec((2000, 128), lambda i, c: (i, 0)),
            pl.BlockSpec((2000, 1), lambda i, c: (i, 0)),
            pl.BlockSpec((1, 128, 128), lambda i, c: (c, 0, 0)),
            pl.BlockSpec((1, 1, 128), lambda i, c: (c, 0, 0)),
        ],
        out_specs=pl.BlockSpec((1, 2000, 128), lambda i, c: (c, i, 0)),
        out_shape=jax.ShapeDtypeStruct((2, N, 128), jnp.float32),
    )(acc, xpp, dis, W0r, b0r)


def _tail_body(acc_ref, h1_ref, dis_ref, w_ref, b_ref, batch_ref, wl_ref,
               bl_ref, w0_ref, b0_ref, w1_ref, b1_ref, wo_ref, bo_ref,
               out_ref, sums_ref, cnts_ref):
    i = pl.program_id(0)
    dis = dis_ref[...]
    m = b_ref[...]
    for c in range(2):
        p = (acc_ref[c] + h1_ref[c]) * dis
        m = m + jnp.dot(p, w_ref[c], preferred_element_type=jnp.float32)
    h = jnp.maximum(m, 0.0)
    h = jnp.maximum(jnp.dot(h, wl_ref[...],
                            preferred_element_type=jnp.float32) + bl_ref[...], 0.0)
    h = jnp.maximum(jnp.dot(h, w0_ref[...],
                            preferred_element_type=jnp.float32) + b0_ref[...], 0.0)
    h = jnp.maximum(jnp.dot(h, w1_ref[...],
                            preferred_element_type=jnp.float32) + b1_ref[...], 0.0)
    seg = lax.broadcasted_iota(jnp.int32, (1, G), 1)
    P = (batch_ref[...] == seg).astype(jnp.float32)
    ps = lax.dot_general(P, h, (((0,), (0,)), ((), ())),
                         preferred_element_type=jnp.float32)
    ones = jnp.ones((h.shape[0], 1), dtype=jnp.float32)
    pc = lax.dot_general(P, ones, (((0,), (0,)), ((), ())),
                         preferred_element_type=jnp.float32)

    @pl.when(i == 0)
    def _():
        sums_ref[...] = ps
        cnts_ref[...] = pc

    @pl.when(i > 0)
    def _():
        sums_ref[...] += ps
        cnts_ref[...] += pc

    @pl.when(i == 4)
    def _():
        pooled = sums_ref[...] / jnp.maximum(cnts_ref[...], 1.0)
        o = jnp.dot(pooled, wo_ref[...],
                    preferred_element_type=jnp.float32) + bo_ref[...]
        out_ref[...] = jax.nn.sigmoid(o)


def _tail(acc2, h1pp, dis, W1r, b1r, batch2d, W_lin, b_lin2, W_l0, b_l02,
          W_l1p, b_l1p, W_outp, b_out2):
    return pl.pallas_call(
        _tail_body,
        grid=(5,),
        in_specs=[
            pl.BlockSpec((2, 2000, 128), lambda i: (0, i, 0)),
            pl.BlockSpec((2, 2000, 128), lambda i: (0, i, 0)),
            pl.BlockSpec((2000, 1), lambda i: (i, 0)),
            pl.BlockSpec((2, 128, 384), lambda i: (0, 0, 0)),
            pl.BlockSpec((1, 384), lambda i: (0, 0)),
            pl.BlockSpec((2000, 1), lambda i: (i, 0)),
            pl.BlockSpec((384, 512), lambda i: (0, 0)),
            pl.BlockSpec((1, 512), lambda i: (0, 0)),
            pl.BlockSpec((512, 256), lambda i: (0, 0)),
            pl.BlockSpec((1, 256), lambda i: (0, 0)),
            pl.BlockSpec((256, 256), lambda i: (0, 0)),
            pl.BlockSpec((1, 256), lambda i: (0, 0)),
            pl.BlockSpec((256, 1), lambda i: (0, 0)),
            pl.BlockSpec((1, 1), lambda i: (0, 0)),
        ],
        out_specs=pl.BlockSpec((G, 1), lambda i: (0, 0)),
        out_shape=jax.ShapeDtypeStruct((G, 1), jnp.float32),
        scratch_shapes=[
            pltpu.VMEM((G, 256), jnp.float32),
            pltpu.VMEM((G, 1), jnp.float32),
        ],
    )(acc2, h1pp, dis, W1r, b1r, batch2d, W_lin, b_lin2, W_l0, b_l02,
      W_l1p, b_l1p, W_outp, b_out2)


def kernel(x, edge_index, edge_weight, batch, W_conv0, b_conv0, W_conv1,
           b_conv1, W_lin, b_lin, W_l0, b_l0, W_l1, b_l1, W_out, b_out):
    pad = 7680
    src = jnp.concatenate([edge_index[0].astype(jnp.int32),
                           (jnp.arange(pad, dtype=jnp.int32) * 4) % N])
    dst = jnp.concatenate([edge_index[1].astype(jnp.int32),
                           (jnp.arange(pad, dtype=jnp.int32) * 4) % N])
    w = jnp.concatenate([edge_weight, jnp.zeros((pad,), jnp.float32)])

    deg_part = jnp.stack(_deg(dst.reshape(32, 128, K), w.reshape(32, 128, K)))
    dis, xpp = _prep(deg_part, x)

    acc1 = _prop(xpp, src.reshape(32, 16, 8, K), dst.reshape(32, 16, 8, K),
                 w.reshape(32, 16, 8, K))

    W0r = W_conv0.reshape(128, 2, 128).transpose(1, 0, 2)
    b0r = b_conv0.reshape(2, 1, 128)
    h1pp = _layer1(acc1, xpp, dis, W0r, b0r)

    src_l2 = jnp.stack([src, src + N]).reshape(2, 16, 32, 8, K)
    q2 = _prop(h1pp.reshape(2 * N, 128), src_l2,
               dst.reshape(16, 32, 8, K), w.reshape(16, 32, 8, K))

    W1r = W_conv1.reshape(2, 128, 384)
    W_l1p = jnp.pad(W_l1, ((0, 0), (0, 86)))
    b_l1p = jnp.pad(b_l1, (0, 86)).reshape(1, 256)
    W_outp = jnp.pad(W_out, ((0, 86), (0, 0)))
    out = _tail(q2, h1pp, dis, W1r, b_conv1.reshape(1, 384),
                batch.astype(jnp.int32).reshape(N, 1), W_lin,
                b_lin.reshape(1, 512), W_l0, b_l0.reshape(1, 256),
                W_l1p, b_l1p, W_outp, b_out.reshape(1, 1))
    return out
